# Initial kernel scaffold; baseline (speedup 1.0000x reference)
import jax
import jax.numpy as jnp
from jax.experimental import pallas as pl


def kernel(x, indices, values, size, W1, b1, W2, b2, edge_weights):
    raise NotImplementedError("write your pallas kernel here")


# speedup vs baseline: 7.4564x; 7.4564x over previous
import jax
import jax.numpy as jnp
from jax import lax
from jax.experimental import pallas as pl
from jax.experimental.pallas import tpu as pltpu
from jax.experimental.pallas import tpu_sc as plsc

_NSC = 2
_NTEC = 16
_NW = _NSC * _NTEC
_L = 16
_CHK = 80


def _tc_dense1(x, W1, b1, ew2, row2, size_arr):
    N, _ = x.shape
    H = W1.shape[1]
    R, Lw = ew2.shape

    def body(x_ref, w_ref, b_ref, ew_ref, row_ref, size_ref, sup_ref, s2_ref):
        sup_ref[...] = (
            jnp.dot(x_ref[...], w_ref[...], preferred_element_type=jnp.float32)
            + b_ref[...]
        )
        sn = jnp.sin(ew_ref[...])
        s2_ref[...] = jnp.where(row_ref[...] < size_ref[0, 0], sn * sn, 0.0)

    return pl.pallas_call(
        body,
        out_shape=[
            jax.ShapeDtypeStruct((N, H), jnp.float32),
            jax.ShapeDtypeStruct((R, Lw), jnp.float32),
        ],
        in_specs=[pl.BlockSpec(memory_space=pltpu.VMEM)] * 5
        + [pl.BlockSpec(memory_space=pltpu.SMEM)],
        out_specs=[pl.BlockSpec(memory_space=pltpu.VMEM)] * 2,
    )(x, W1, b1, ew2, row2, size_arr)


def _tc_dense2(p0, p1, W2, b2):
    N, _ = p0.shape
    C = W2.shape[1]

    def body(p0_ref, p1_ref, w_ref, b_ref, out_ref):
        h = jnp.maximum(p0_ref[...] + p1_ref[...], 0.0)
        out_ref[...] = (
            jnp.dot(h, w_ref[...], preferred_element_type=jnp.float32) + b_ref[...]
        )

    return pl.pallas_call(
        body, out_shape=jax.ShapeDtypeStruct((N, C), jnp.float32)
    )(p0, p1, W2, b2)


def _tc_logsoftmax(p0, p1):
    N, C = p0.shape

    def body(p0_ref, p1_ref, out_ref):
        z = p0_ref[...] + p1_ref[...]
        m = jnp.max(z, axis=1, keepdims=True)
        e = jnp.exp(z - m)
        lse = jnp.log(jnp.sum(e, axis=1, keepdims=True))
        out_ref[...] = (z - m) - lse

    return pl.pallas_call(
        body, out_shape=jax.ShapeDtypeStruct((N, C), jnp.float32)
    )(p0, p1)


def _sc_layer1(s2, row, col, sup):
    E = s2.shape[0]
    N, H = sup.shape
    NG = H // _L
    EPT = E // _NW
    NCHK = EPT // _CHK
    EPS = E // _NTEC
    DCHK = 2000
    NDC = EPS // DCHK
    NPT = (N // _NTEC) // 8 * 8
    TAIL = N - _NTEC * NPT
    NP = ((N + _L - 1) // _L + _NTEC - 1) // _NTEC * _NTEC * _L
    NSL = NP // _NTEC

    def body(s2_h, row_h, col_h, sup_h, vals_h, part_h,
             denom_v, recip_v, tmp_v, accd_v, rowd_v, sd_v,
             colb, rowb, sb, valsb, rows0, sem0, stage_sp, recip_sp, acc_sp):
        c = lax.axis_index("c")
        s = lax.axis_index("s")
        wid = c * _NTEC + s
        zero16 = jnp.zeros((_L,), jnp.float32)

        def zden(i, carry):
            denom_v[pl.ds(i * _L, _L)] = zero16
            return carry
        lax.fori_loop(0, NP // _L, zden, 0)

        def zrow(i, carry):
            for j in range(NG):
                rows0[i, pl.ds(j * _L, _L)] = zero16
            return carry
        lax.fori_loop(0, _CHK, zrow, 0)

        rbase = s * NPT
        off = 0
        for _i in range(NPT // _CHK):
            pltpu.sync_copy(rows0, acc_sp.at[pl.ds(rbase + off, _CHK)])
            off += _CHK
        rem = NPT - off
        if rem:
            pltpu.sync_copy(rows0.at[pl.ds(0, rem)],
                            acc_sp.at[pl.ds(rbase + off, rem)])
        if TAIL:
            @pl.when(s == 0)
            def _():
                pltpu.sync_copy(rows0.at[pl.ds(0, TAIL)],
                                acc_sp.at[pl.ds(N - TAIL, TAIL)])

        dbase = s * EPS
        for g in range(NDC):
            pltpu.sync_copy(row_h.at[pl.ds(dbase + g * DCHK, DCHK)], rowd_v)
            pltpu.sync_copy(s2_h.at[pl.ds(dbase + g * DCHK, DCHK)], sd_v)

            def dacc(i, carry):
                idx = rowd_v[pl.ds(i * _L, _L)]
                sv = sd_v[pl.ds(i * _L, _L)]
                plsc.addupdate_scatter(denom_v, [idx], sv)
                return carry
            lax.fori_loop(0, DCHK // _L, dacc, 0)

        pltpu.sync_copy(denom_v, stage_sp.at[s])
        plsc.subcore_barrier()

        def zacc(i, carry):
            accd_v[pl.ds(i * _L, _L)] = zero16
            return carry
        lax.fori_loop(0, NSL // _L, zacc, 0)
        for k in range(_NTEC):
            pltpu.sync_copy(stage_sp.at[k, pl.ds(s * NSL, NSL)], tmp_v)

            def radd(i, carry):
                sl = pl.ds(i * _L, _L)
                accd_v[sl] = accd_v[sl] + tmp_v[sl]
                return carry
            lax.fori_loop(0, NSL // _L, radd, 0)

        def rcp(i, carry):
            sl = pl.ds(i * _L, _L)
            accd_v[sl] = 1.0 / (accd_v[sl] + 1e-12)
            return carry
        lax.fori_loop(0, NSL // _L, rcp, 0)
        pltpu.sync_copy(accd_v, recip_sp.at[pl.ds(s * NSL, NSL)])
        plsc.subcore_barrier()
        pltpu.sync_copy(recip_sp, recip_v)

        ebase = wid * EPT

        def chunk(g, carry):
            eoff = ebase + g * _CHK
            pltpu.sync_copy(col_h.at[pl.ds(eoff, _CHK)], colb)
            cp = pltpu.async_copy(sup_h.at[colb], rows0, sem0)
            pltpu.sync_copy(row_h.at[pl.ds(eoff, _CHK)], rowb)
            pltpu.sync_copy(s2_h.at[pl.ds(eoff, _CHK)], sb)

            def vgrp(i, carry2):
                idx = rowb[pl.ds(i * _L, _L)]
                r = plsc.load_gather(recip_v, [idx])
                valsb[pl.ds(i * _L, _L)] = sb[pl.ds(i * _L, _L)] * r
                return carry2
            lax.fori_loop(0, _CHK // _L, vgrp, 0)
            pltpu.sync_copy(valsb, vals_h.at[pl.ds(eoff, _CHK)])
            cp.wait()

            def scale(e, carry2):
                vsp = plsc.load_gather(valsb, [lax.broadcast(e, (_L,))])
                for j in range(NG):
                    sl = pl.ds(j * _L, _L)
                    rows0[e, sl] = rows0[e, sl] * vsp
                return carry2
            lax.fori_loop(0, _CHK, scale, 0)
            pltpu.sync_copy(rows0, acc_sp.at[rowb], add=True)
            return carry
        lax.fori_loop(0, NCHK, chunk, 0)

        plsc.subcore_barrier()
        pltpu.sync_copy(acc_sp.at[pl.ds(rbase, NPT)],
                        part_h.at[c, pl.ds(rbase, NPT)])
        if TAIL:
            @pl.when(s == 0)
            def _():
                pltpu.sync_copy(acc_sp.at[pl.ds(N - TAIL, TAIL)],
                                part_h.at[c, pl.ds(N - TAIL, TAIL)])

    mesh = plsc.VectorSubcoreMesh(
        core_axis_name="c", subcore_axis_name="s",
        num_cores=_NSC, num_subcores=_NTEC)
    fn = pl.kernel(
        body,
        out_type=[
            jax.ShapeDtypeStruct((E,), jnp.float32),
            jax.ShapeDtypeStruct((_NSC, N, H), jnp.float32),
        ],
        mesh=mesh,
        scratch_types=[
            pltpu.VMEM((NP,), jnp.float32),
            pltpu.VMEM((NP,), jnp.float32),
            pltpu.VMEM((NSL,), jnp.float32),
            pltpu.VMEM((NSL,), jnp.float32),
            pltpu.VMEM((DCHK,), jnp.int32),
            pltpu.VMEM((DCHK,), jnp.float32),
            pltpu.VMEM((_CHK,), jnp.int32),
            pltpu.VMEM((_CHK,), jnp.int32),
            pltpu.VMEM((_CHK,), jnp.float32),
            pltpu.VMEM((_CHK,), jnp.float32),
            pltpu.VMEM((_CHK, H), jnp.float32),
            pltpu.SemaphoreType.DMA,
            pltpu.VMEM_SHARED((_NTEC, NP), jnp.float32),
            pltpu.VMEM_SHARED((NP,), jnp.float32),
            pltpu.VMEM_SHARED((N, H), jnp.float32),
        ],
        compiler_params=pltpu.CompilerParams(needs_layout_passes=False),
    )
    return fn(s2, row, col, sup)


def _sc_layer2(vals, row, col, sup):
    E = vals.shape[0]
    N, C = sup.shape
    NG = C // _L
    EPT = E // _NW
    NCHK = EPT // _CHK
    NPT = (N // _NTEC) // 8 * 8
    TAIL = N - _NTEC * NPT

    def body(vals_h, row_h, col_h, sup_h, part_h,
             colb, rowb, valsb, rows0, sem0, acc_sp):
        c = lax.axis_index("c")
        s = lax.axis_index("s")
        wid = c * _NTEC + s
        zero16 = jnp.zeros((_L,), jnp.float32)

        def zrow(i, carry):
            for j in range(NG):
                rows0[i, pl.ds(j * _L, _L)] = zero16
            return carry
        lax.fori_loop(0, _CHK, zrow, 0)

        rbase = s * NPT
        off = 0
        for _i in range(NPT // _CHK):
            pltpu.sync_copy(rows0, acc_sp.at[pl.ds(rbase + off, _CHK)])
            off += _CHK
        rem = NPT - off
        if rem:
            pltpu.sync_copy(rows0.at[pl.ds(0, rem)],
                            acc_sp.at[pl.ds(rbase + off, rem)])
        if TAIL:
            @pl.when(s == 0)
            def _():
                pltpu.sync_copy(rows0.at[pl.ds(0, TAIL)],
                                acc_sp.at[pl.ds(N - TAIL, TAIL)])

        plsc.subcore_barrier()

        ebase = wid * EPT

        def chunk(g, carry):
            eoff = ebase + g * _CHK
            pltpu.sync_copy(col_h.at[pl.ds(eoff, _CHK)], colb)
            cp = pltpu.async_copy(sup_h.at[colb], rows0, sem0)
            pltpu.sync_copy(row_h.at[pl.ds(eoff, _CHK)], rowb)
            pltpu.sync_copy(vals_h.at[pl.ds(eoff, _CHK)], valsb)
            cp.wait()

            def scale(e, carry2):
                vsp = plsc.load_gather(valsb, [lax.broadcast(e, (_L,))])
                for j in range(NG):
                    sl = pl.ds(j * _L, _L)
                    rows0[e, sl] = rows0[e, sl] * vsp
                return carry2
            lax.fori_loop(0, _CHK, scale, 0)
            pltpu.sync_copy(rows0, acc_sp.at[rowb], add=True)
            return carry
        lax.fori_loop(0, NCHK, chunk, 0)

        plsc.subcore_barrier()
        pltpu.sync_copy(acc_sp.at[pl.ds(rbase, NPT)],
                        part_h.at[c, pl.ds(rbase, NPT)])
        if TAIL:
            @pl.when(s == 0)
            def _():
                pltpu.sync_copy(acc_sp.at[pl.ds(N - TAIL, TAIL)],
                                part_h.at[c, pl.ds(N - TAIL, TAIL)])

    mesh = plsc.VectorSubcoreMesh(
        core_axis_name="c", subcore_axis_name="s",
        num_cores=_NSC, num_subcores=_NTEC)
    fn = pl.kernel(
        body,
        out_type=jax.ShapeDtypeStruct((_NSC, N, C), jnp.float32),
        mesh=mesh,
        scratch_types=[
            pltpu.VMEM((_CHK,), jnp.int32),
            pltpu.VMEM((_CHK,), jnp.int32),
            pltpu.VMEM((_CHK,), jnp.float32),
            pltpu.VMEM((_CHK, C), jnp.float32),
            pltpu.SemaphoreType.DMA,
            pltpu.VMEM_SHARED((N, C), jnp.float32),
        ],
        compiler_params=pltpu.CompilerParams(
            needs_layout_passes=False, use_tc_tiling_on_sc=False),
    )
    return fn(vals, row, col, sup)


def kernel(x, indices, values, size, W1, b1, W2, b2, edge_weights):
    N, _ = x.shape
    E = edge_weights.shape[0]
    H = W1.shape[1]
    C = W2.shape[1]
    row = indices[0]
    col = indices[1]
    ew2 = edge_weights.reshape(E // 128, 128)
    row2 = row.reshape(E // 128, 128)
    size_arr = jnp.full((1, 1), size, jnp.int32)

    sup1, s2 = _tc_dense1(x, W1, b1.reshape(1, H), ew2, row2, size_arr)
    vals, part1 = _sc_layer1(s2.reshape(E), row, col, sup1)
    sup2 = _tc_dense2(part1[0], part1[1], W2, b2.reshape(1, C))
    part2 = _sc_layer2(vals, row, col, sup2)
    return _tc_logsoftmax(part2[0], part2[1])


# speedup vs baseline: 7.6215x; 1.0221x over previous
import jax
import jax.numpy as jnp
from jax import lax
from jax.experimental import pallas as pl
from jax.experimental.pallas import tpu as pltpu
from jax.experimental.pallas import tpu_sc as plsc

_NSC = 2
_NTEC = 16
_NW = _NSC * _NTEC
_L = 16
_CHK = 128


def _tc_dense1(x, W1, b1, ew2, row2, size_arr):
    N, _ = x.shape
    H = W1.shape[1]
    R, Lw = ew2.shape

    def body(x_ref, w_ref, b_ref, ew_ref, row_ref, size_ref, sup_ref, s2_ref):
        sup_ref[...] = (
            jnp.dot(x_ref[...], w_ref[...], preferred_element_type=jnp.float32)
            + b_ref[...]
        )
        sn = jnp.sin(ew_ref[...])
        s2_ref[...] = jnp.where(row_ref[...] < size_ref[0, 0], sn * sn, 0.0)

    return pl.pallas_call(
        body,
        out_shape=[
            jax.ShapeDtypeStruct((N, H), jnp.float32),
            jax.ShapeDtypeStruct((R, Lw), jnp.float32),
        ],
        in_specs=[pl.BlockSpec(memory_space=pltpu.VMEM)] * 5
        + [pl.BlockSpec(memory_space=pltpu.SMEM)],
        out_specs=[pl.BlockSpec(memory_space=pltpu.VMEM)] * 2,
    )(x, W1, b1, ew2, row2, size_arr)


def _tc_dense2(p0, p1, W2, b2):
    N, _ = p0.shape
    C = W2.shape[1]

    def body(p0_ref, p1_ref, w_ref, b_ref, out_ref):
        h = jnp.maximum(p0_ref[...] + p1_ref[...], 0.0)
        out_ref[...] = (
            jnp.dot(h, w_ref[...], preferred_element_type=jnp.float32) + b_ref[...]
        )

    return pl.pallas_call(
        body, out_shape=jax.ShapeDtypeStruct((N, C), jnp.float32)
    )(p0, p1, W2, b2)


def _tc_logsoftmax(p0, p1):
    N, C = p0.shape

    def body(p0_ref, p1_ref, out_ref):
        z = p0_ref[...] + p1_ref[...]
        m = jnp.max(z, axis=1, keepdims=True)
        e = jnp.exp(z - m)
        lse = jnp.log(jnp.sum(e, axis=1, keepdims=True))
        out_ref[...] = (z - m) - lse

    return pl.pallas_call(
        body, out_shape=jax.ShapeDtypeStruct((N, C), jnp.float32)
    )(p0, p1)


def _zero_rows(rows, ng, chk):
    zero16 = jnp.zeros((_L,), jnp.float32)

    def zrow(i, carry):
        for j in range(ng):
            rows[i, pl.ds(j * _L, _L)] = zero16
        return carry
    lax.fori_loop(0, chk, zrow, 0)


def _zero_acc_slice(rows, acc_sp, s, npt, tail, n, chk):
    rbase = s * npt
    off = 0
    for _i in range(npt // chk):
        pltpu.sync_copy(rows, acc_sp.at[pl.ds(rbase + off, chk)])
        off += chk
    rem = npt - off
    if rem:
        pltpu.sync_copy(rows.at[pl.ds(0, rem)],
                        acc_sp.at[pl.ds(rbase + off, rem)])
    if tail:
        @pl.when(s == 0)
        def _():
            pltpu.sync_copy(rows.at[pl.ds(0, tail)],
                            acc_sp.at[pl.ds(n - tail, tail)])


def _copy_out_slice(acc_sp, part_h, c, s, npt, tail, n):
    rbase = s * npt
    pltpu.sync_copy(acc_sp.at[pl.ds(rbase, npt)],
                    part_h.at[c, pl.ds(rbase, npt)])
    if tail:
        @pl.when(s == 0)
        def _():
            pltpu.sync_copy(acc_sp.at[pl.ds(n - tail, tail)],
                            part_h.at[c, pl.ds(n - tail, tail)])


def _scale_rows(rows, valsb, ng, chk):
    def sbody(i, carry):
        for u in range(4):
            e = i * 4 + u
            vsp = plsc.load_gather(valsb, [lax.broadcast(e, (_L,))])
            for j in range(ng):
                sl = pl.ds(j * _L, _L)
                rows[e, sl] = rows[e, sl] * vsp
        return carry
    lax.fori_loop(0, chk // 4, sbody, 0)


def _sc_layer1(s2, row, col, sup):
    E = s2.shape[0]
    N, H = sup.shape
    NG = H // _L
    EPS = E // _NTEC
    DCHK = 2000
    NDC = EPS // DCHK
    NPT = (N // _NTEC) // 8 * 8
    TAIL = N - _NTEC * NPT
    NP = ((N + _L - 1) // _L + _NTEC - 1) // _NTEC * _NTEC * _L
    NSL = NP // _NTEC
    CHK1 = 80
    KPT = E // (_NW * CHK1)
    PAIRS = KPT // 2

    def body(s2_h, row_h, col_h, sup_h, vals_h, part_h,
             denom_v, recip_v, tmp_v, accd_v, rowd_v, sd_v,
             colA, rowA, sA, valsb, rowsA, semA, stage_sp, recip_sp, acc_sp):
        c = lax.axis_index("c")
        s = lax.axis_index("s")
        wid = c * _NTEC + s
        zero16 = jnp.zeros((_L,), jnp.float32)

        def zden(i, carry):
            denom_v[pl.ds(i * _L, _L)] = zero16
            return carry
        lax.fori_loop(0, NP // _L, zden, 0)


        _zero_rows(rowsA, NG, CHK1)
        _zero_acc_slice(rowsA, acc_sp, s, NPT, TAIL, N, CHK1)

        dbase = s * EPS
        for g in range(NDC):
            pltpu.sync_copy(row_h.at[pl.ds(dbase + g * DCHK, DCHK)], rowd_v)
            pltpu.sync_copy(s2_h.at[pl.ds(dbase + g * DCHK, DCHK)], sd_v)

            def dacc(i, carry):
                for u in range(5):
                    sl = pl.ds((i * 5 + u) * _L, _L)
                    plsc.addupdate_scatter(denom_v, [rowd_v[sl]], sd_v[sl])
                return carry
            lax.fori_loop(0, DCHK // (5 * _L), dacc, 0)

        pltpu.sync_copy(denom_v, stage_sp.at[s])
        plsc.subcore_barrier()

        def zacc(i, carry):
            accd_v[pl.ds(i * _L, _L)] = zero16
            return carry
        lax.fori_loop(0, NSL // _L, zacc, 0)
        for k in range(_NTEC):
            pltpu.sync_copy(stage_sp.at[k, pl.ds(s * NSL, NSL)], tmp_v)

            def radd(i, carry):
                sl = pl.ds(i * _L, _L)
                accd_v[sl] = accd_v[sl] + tmp_v[sl]
                return carry
            lax.fori_loop(0, NSL // _L, radd, 0)

        def rcp(i, carry):
            sl = pl.ds(i * _L, _L)
            accd_v[sl] = 1.0 / (accd_v[sl] + 1e-12)
            return carry
        lax.fori_loop(0, NSL // _L, rcp, 0)
        pltpu.sync_copy(accd_v, recip_sp.at[pl.ds(s * NSL, NSL)])
        plsc.subcore_barrier()
        pltpu.sync_copy(recip_sp, recip_v)

        def eoff_of(k):
            return (wid + k * _NW) * CHK1

        def compute_vals(k_idx, rowX, sX):
            pltpu.sync_copy(row_h.at[pl.ds(eoff_of(k_idx), CHK1)], rowX)
            pltpu.sync_copy(s2_h.at[pl.ds(eoff_of(k_idx), CHK1)], sX)
            for i in range(CHK1 // _L):
                sl = pl.ds(i * _L, _L)
                r = plsc.load_gather(recip_v, [rowX[sl]])
                valsb[sl] = sX[sl] * r
            pltpu.sync_copy(valsb, vals_h.at[pl.ds(eoff_of(k_idx), CHK1)])

        def chunk(k, carry):
            pltpu.sync_copy(col_h.at[pl.ds(eoff_of(k), CHK1)], colA)
            gA = pltpu.async_copy(sup_h.at[colA], rowsA, semA)
            compute_vals(k, rowA, sA)
            gA.wait()
            _scale_rows(rowsA, valsb, NG, CHK1)
            pltpu.sync_copy(rowsA, acc_sp.at[rowA], add=True)
            return carry
        lax.fori_loop(0, KPT, chunk, 0)

        plsc.subcore_barrier()
        _copy_out_slice(acc_sp, part_h, c, s, NPT, TAIL, N)

    mesh = plsc.VectorSubcoreMesh(
        core_axis_name="c", subcore_axis_name="s",
        num_cores=_NSC, num_subcores=_NTEC)
    fn = pl.kernel(
        body,
        out_type=[
            jax.ShapeDtypeStruct((E,), jnp.float32),
            jax.ShapeDtypeStruct((_NSC, N, H), jnp.float32),
        ],
        mesh=mesh,
        scratch_types=[
            pltpu.VMEM((NP,), jnp.float32),
            pltpu.VMEM((NP,), jnp.float32),
            pltpu.VMEM((NSL,), jnp.float32),
            pltpu.VMEM((NSL,), jnp.float32),
            pltpu.VMEM((DCHK,), jnp.int32),
            pltpu.VMEM((DCHK,), jnp.float32),
            pltpu.VMEM((CHK1,), jnp.int32),
            pltpu.VMEM((CHK1,), jnp.int32),
            pltpu.VMEM((CHK1,), jnp.float32),
            pltpu.VMEM((CHK1,), jnp.float32),
            pltpu.VMEM((CHK1, H), jnp.float32),
            pltpu.SemaphoreType.DMA,
            pltpu.VMEM_SHARED((_NTEC, NP), jnp.float32),
            pltpu.VMEM_SHARED((NP,), jnp.float32),
            pltpu.VMEM_SHARED((N, H), jnp.float32),
        ],
        compiler_params=pltpu.CompilerParams(needs_layout_passes=False),
    )
    return fn(s2, row, col, sup)


def _sc_layer2(vals, row, col, sup):
    E = vals.shape[0]
    N, C = sup.shape
    NG = C // _L
    NPT = (N // _NTEC) // 8 * 8
    TAIL = N - _NTEC * NPT
    CHK2 = 80
    KPT = E // (_NW * CHK2)
    PAIRS = KPT // 2

    def body(vals_h, row_h, col_h, sup_h, part_h,
             colA, rowA, valsb, rowsA, semA, acc_sp):
        c = lax.axis_index("c")
        s = lax.axis_index("s")
        wid = c * _NTEC + s

        _zero_rows(rowsA, NG, CHK2)
        _zero_acc_slice(rowsA, acc_sp, s, NPT, TAIL, N, CHK2)
        plsc.subcore_barrier()

        def eoff_of(k):
            return (wid + k * _NW) * CHK2

        def stage_rv(k_idx, rowX):
            pltpu.sync_copy(row_h.at[pl.ds(eoff_of(k_idx), CHK2)], rowX)
            pltpu.sync_copy(vals_h.at[pl.ds(eoff_of(k_idx), CHK2)], valsb)

        def chunk(k, carry):
            pltpu.sync_copy(col_h.at[pl.ds(eoff_of(k), CHK2)], colA)
            gA = pltpu.async_copy(sup_h.at[colA], rowsA, semA)
            stage_rv(k, rowA)
            gA.wait()
            _scale_rows(rowsA, valsb, NG, CHK2)
            pltpu.sync_copy(rowsA, acc_sp.at[rowA], add=True)
            return carry
        lax.fori_loop(0, KPT, chunk, 0)

        plsc.subcore_barrier()
        _copy_out_slice(acc_sp, part_h, c, s, NPT, TAIL, N)

    mesh = plsc.VectorSubcoreMesh(
        core_axis_name="c", subcore_axis_name="s",
        num_cores=_NSC, num_subcores=_NTEC)
    fn = pl.kernel(
        body,
        out_type=jax.ShapeDtypeStruct((_NSC, N, C), jnp.float32),
        mesh=mesh,
        scratch_types=[
            pltpu.VMEM((CHK2,), jnp.int32),
            pltpu.VMEM((CHK2,), jnp.int32),
            pltpu.VMEM((CHK2,), jnp.float32),
            pltpu.VMEM((CHK2, C), jnp.float32),
            pltpu.SemaphoreType.DMA,
            pltpu.VMEM_SHARED((N, C), jnp.float32),
        ],
        compiler_params=pltpu.CompilerParams(
            needs_layout_passes=False, use_tc_tiling_on_sc=False),
    )
    return fn(vals, row, col, sup)


def kernel(x, indices, values, size, W1, b1, W2, b2, edge_weights):
    N, _ = x.shape
    E = edge_weights.shape[0]
    H = W1.shape[1]
    C = W2.shape[1]
    row = indices[0]
    col = indices[1]
    ew2 = edge_weights.reshape(E // 128, 128)
    row2 = row.reshape(E // 128, 128)
    size_arr = jnp.full((1, 1), size, jnp.int32)

    sup1, s2 = _tc_dense1(x, W1, b1.reshape(1, H), ew2, row2, size_arr)
    vals, part1 = _sc_layer1(s2.reshape(E), row, col, sup1)
    sup2 = _tc_dense2(part1[0], part1[1], W2, b2.reshape(1, C))
    part2 = _sc_layer2(vals, row, col, sup2)
    return _tc_logsoftmax(part2[0], part2[1])


# speedup vs baseline: 7.9668x; 1.0453x over previous
import jax
import jax.numpy as jnp
from jax import lax
from jax.experimental import pallas as pl
from jax.experimental.pallas import tpu as pltpu
from jax.experimental.pallas import tpu_sc as plsc

_NSC = 2
_NTEC = 16
_NW = _NSC * _NTEC
_L = 16
_CHK = 128


def _tc_dense1(x, W1, b1, ew2, row2, size_arr):
    N, _ = x.shape
    H = W1.shape[1]
    R, Lw = ew2.shape

    def body(x_ref, w_ref, b_ref, ew_ref, row_ref, size_ref, sup_ref, s2_ref):
        sup_ref[...] = (
            jnp.dot(x_ref[...], w_ref[...], preferred_element_type=jnp.float32)
            + b_ref[...]
        )
        sn = jnp.sin(ew_ref[...])
        s2_ref[...] = jnp.where(row_ref[...] < size_ref[0, 0], sn * sn, 0.0)

    return pl.pallas_call(
        body,
        out_shape=[
            jax.ShapeDtypeStruct((N, H), jnp.float32),
            jax.ShapeDtypeStruct((R, Lw), jnp.float32),
        ],
        in_specs=[pl.BlockSpec(memory_space=pltpu.VMEM)] * 5
        + [pl.BlockSpec(memory_space=pltpu.SMEM)],
        out_specs=[pl.BlockSpec(memory_space=pltpu.VMEM)] * 2,
    )(x, W1, b1, ew2, row2, size_arr)


def _tc_dense2(p0, p1, W2, b2):
    N, _ = p0.shape
    C = W2.shape[1]

    def body(p0_ref, p1_ref, w_ref, b_ref, out_ref):
        h = jnp.maximum(p0_ref[...] + p1_ref[...], 0.0)
        out_ref[...] = (
            jnp.dot(h, w_ref[...], preferred_element_type=jnp.float32) + b_ref[...]
        )

    return pl.pallas_call(
        body, out_shape=jax.ShapeDtypeStruct((N, C), jnp.float32)
    )(p0, p1, W2, b2)


def _tc_logsoftmax(p0, p1):
    N, C = p0.shape

    def body(p0_ref, p1_ref, out_ref):
        z = p0_ref[...] + p1_ref[...]
        m = jnp.max(z, axis=1, keepdims=True)
        e = jnp.exp(z - m)
        lse = jnp.log(jnp.sum(e, axis=1, keepdims=True))
        out_ref[...] = (z - m) - lse

    return pl.pallas_call(
        body, out_shape=jax.ShapeDtypeStruct((N, C), jnp.float32)
    )(p0, p1)


def _zero_rows(rows, ng, chk):
    zero16 = jnp.zeros((_L,), jnp.float32)

    def zrow(i, carry):
        for j in range(ng):
            rows[i, pl.ds(j * _L, _L)] = zero16
        return carry
    lax.fori_loop(0, chk, zrow, 0)


def _zero_acc_slice(rows, acc_sp, s, npt, tail, n, chk):
    rbase = s * npt
    off = 0
    for _i in range(npt // chk):
        pltpu.sync_copy(rows, acc_sp.at[pl.ds(rbase + off, chk)])
        off += chk
    rem = npt - off
    if rem:
        pltpu.sync_copy(rows.at[pl.ds(0, rem)],
                        acc_sp.at[pl.ds(rbase + off, rem)])
    if tail:
        @pl.when(s == 0)
        def _():
            pltpu.sync_copy(rows.at[pl.ds(0, tail)],
                            acc_sp.at[pl.ds(n - tail, tail)])


def _copy_out_slice(acc_sp, part_h, c, s, npt, tail, n):
    rbase = s * npt
    pltpu.sync_copy(acc_sp.at[pl.ds(rbase, npt)],
                    part_h.at[c, pl.ds(rbase, npt)])
    if tail:
        @pl.when(s == 0)
        def _():
            pltpu.sync_copy(acc_sp.at[pl.ds(n - tail, tail)],
                            part_h.at[c, pl.ds(n - tail, tail)])


def _scale_rows(rows, valsb, ng, chk):
    def sbody(i, carry):
        for u in range(4):
            e = i * 4 + u
            vsp = plsc.load_gather(valsb, [lax.broadcast(e, (_L,))])
            for j in range(ng):
                sl = pl.ds(j * _L, _L)
                rows[e, sl] = rows[e, sl] * vsp
        return carry
    lax.fori_loop(0, chk // 4, sbody, 0)


def _sc_layer1(s2, row, col, sup):
    E = s2.shape[0]
    N, H = sup.shape
    NG = H // _L
    EPS = E // _NTEC
    DCHK = 800
    NDC = EPS // DCHK
    NPT = (N // _NTEC) // 8 * 8
    TAIL = N - _NTEC * NPT
    NP = ((N + _L - 1) // _L + _NTEC - 1) // _NTEC * _NTEC * _L
    NSL = NP // _NTEC
    CHK1 = 80
    KPT = E // (_NW * CHK1)
    PAIRS = KPT // 2

    def body(s2_h, row_h, col_h, sup_h, vals_h, part_h,
             denom_v, tmp_v, accd_v, rowd_v, sd_v,
             colA, colB, rowA, rowB, sA, sB, valsb, rowsA, rowsB,
             semA, semB, ssA, ssB, stage_sp, recip_sp, acc_sp):
        c = lax.axis_index("c")
        s = lax.axis_index("s")
        wid = c * _NTEC + s
        zero16 = jnp.zeros((_L,), jnp.float32)

        def zden(i, carry):
            denom_v[pl.ds(i * _L, _L)] = zero16
            return carry
        lax.fori_loop(0, NP // _L, zden, 0)


        _zero_rows(rowsA, NG, CHK1)
        _zero_acc_slice(rowsA, acc_sp, s, NPT, TAIL, N, CHK1)

        dbase = s * EPS
        for g in range(NDC):
            pltpu.sync_copy(row_h.at[pl.ds(dbase + g * DCHK, DCHK)], rowd_v)
            pltpu.sync_copy(s2_h.at[pl.ds(dbase + g * DCHK, DCHK)], sd_v)

            def dacc(i, carry):
                for u in range(5):
                    sl = pl.ds((i * 5 + u) * _L, _L)
                    plsc.addupdate_scatter(denom_v, [rowd_v[sl]], sd_v[sl])
                return carry
            lax.fori_loop(0, DCHK // (5 * _L), dacc, 0)

        pltpu.sync_copy(denom_v, stage_sp.at[s])
        plsc.subcore_barrier()

        def zacc(i, carry):
            accd_v[pl.ds(i * _L, _L)] = zero16
            return carry
        lax.fori_loop(0, NSL // _L, zacc, 0)
        for k in range(_NTEC):
            pltpu.sync_copy(stage_sp.at[k, pl.ds(s * NSL, NSL)], tmp_v)

            def radd(i, carry):
                sl = pl.ds(i * _L, _L)
                accd_v[sl] = accd_v[sl] + tmp_v[sl]
                return carry
            lax.fori_loop(0, NSL // _L, radd, 0)

        def rcp(i, carry):
            sl = pl.ds(i * _L, _L)
            accd_v[sl] = 1.0 / (accd_v[sl] + 1e-12)
            return carry
        lax.fori_loop(0, NSL // _L, rcp, 0)
        pltpu.sync_copy(accd_v, recip_sp.at[pl.ds(s * NSL, NSL)])
        plsc.subcore_barrier()
        pltpu.sync_copy(recip_sp, denom_v)

        def eoff_of(k):
            return (wid + k * _NW) * CHK1

        def compute_vals(k_idx, rowX, sX):
            pltpu.sync_copy(row_h.at[pl.ds(eoff_of(k_idx), CHK1)], rowX)
            pltpu.sync_copy(s2_h.at[pl.ds(eoff_of(k_idx), CHK1)], sX)
            for i in range(CHK1 // _L):
                sl = pl.ds(i * _L, _L)
                r = plsc.load_gather(denom_v, [rowX[sl]])
                valsb[sl] = sX[sl] * r
            pltpu.sync_copy(valsb, vals_h.at[pl.ds(eoff_of(k_idx), CHK1)])

        def pair(i, carry):
            kA = 2 * i
            kB = 2 * i + 1
            pltpu.sync_copy(col_h.at[pl.ds(eoff_of(kA), CHK1)], colA)
            gA = pltpu.async_copy(sup_h.at[colA], rowsA, semA)
            pltpu.sync_copy(col_h.at[pl.ds(eoff_of(kB), CHK1)], colB)
            gB = pltpu.async_copy(sup_h.at[colB], rowsB, semB)
            compute_vals(kA, rowA, sA)
            gA.wait()
            _scale_rows(rowsA, valsb, NG, CHK1)
            scatA = pltpu.async_copy(rowsA, acc_sp.at[rowA], ssA, add=True)
            compute_vals(kB, rowB, sB)
            gB.wait()
            _scale_rows(rowsB, valsb, NG, CHK1)
            scatA.wait()
            scatB = pltpu.async_copy(rowsB, acc_sp.at[rowB], ssB, add=True)
            scatB.wait()
            return carry
        lax.fori_loop(0, KPT // 2, pair, 0)

        if KPT % 2:
            k = KPT - 1
            pltpu.sync_copy(col_h.at[pl.ds(eoff_of(k), CHK1)], colA)
            gA = pltpu.async_copy(sup_h.at[colA], rowsA, semA)
            compute_vals(k, rowA, sA)
            gA.wait()
            _scale_rows(rowsA, valsb, NG, CHK1)
            pltpu.sync_copy(rowsA, acc_sp.at[rowA], add=True)

        plsc.subcore_barrier()
        _copy_out_slice(acc_sp, part_h, c, s, NPT, TAIL, N)

    mesh = plsc.VectorSubcoreMesh(
        core_axis_name="c", subcore_axis_name="s",
        num_cores=_NSC, num_subcores=_NTEC)
    fn = pl.kernel(
        body,
        out_type=[
            jax.ShapeDtypeStruct((E,), jnp.float32),
            jax.ShapeDtypeStruct((_NSC, N, H), jnp.float32),
        ],
        mesh=mesh,
        scratch_types=[
            pltpu.VMEM((NP,), jnp.float32),
            pltpu.VMEM((NSL,), jnp.float32),
            pltpu.VMEM((NSL,), jnp.float32),
            pltpu.VMEM((DCHK,), jnp.int32),
            pltpu.VMEM((DCHK,), jnp.float32),
            pltpu.VMEM((CHK1,), jnp.int32),
            pltpu.VMEM((CHK1,), jnp.int32),
            pltpu.VMEM((CHK1,), jnp.int32),
            pltpu.VMEM((CHK1,), jnp.int32),
            pltpu.VMEM((CHK1,), jnp.float32),
            pltpu.VMEM((CHK1,), jnp.float32),
            pltpu.VMEM((CHK1,), jnp.float32),
            pltpu.VMEM((CHK1, H), jnp.float32),
            pltpu.VMEM((CHK1, H), jnp.float32),
            pltpu.SemaphoreType.DMA,
            pltpu.SemaphoreType.DMA,
            pltpu.SemaphoreType.DMA,
            pltpu.SemaphoreType.DMA,
            pltpu.VMEM_SHARED((_NTEC, NP), jnp.float32),
            pltpu.VMEM_SHARED((NP,), jnp.float32),
            pltpu.VMEM_SHARED((N, H), jnp.float32),
        ],
        compiler_params=pltpu.CompilerParams(needs_layout_passes=False),
    )
    return fn(s2, row, col, sup)


def _sc_layer2(vals, row, col, sup):
    E = vals.shape[0]
    N, C = sup.shape
    NG = C // _L
    NPT = (N // _NTEC) // 8 * 8
    TAIL = N - _NTEC * NPT
    CHK2 = 80
    KPT = E // (_NW * CHK2)
    PAIRS = KPT // 2

    def body(vals_h, row_h, col_h, sup_h, part_h,
             colA, colB, rowA, rowB, valsb, rowsA, rowsB,
             semA, semB, ssA, ssB, acc_sp):
        c = lax.axis_index("c")
        s = lax.axis_index("s")
        wid = c * _NTEC + s

        _zero_rows(rowsA, NG, CHK2)
        _zero_acc_slice(rowsA, acc_sp, s, NPT, TAIL, N, CHK2)
        plsc.subcore_barrier()

        def eoff_of(k):
            return (wid + k * _NW) * CHK2

        def stage_rv(k_idx, rowX):
            pltpu.sync_copy(row_h.at[pl.ds(eoff_of(k_idx), CHK2)], rowX)
            pltpu.sync_copy(vals_h.at[pl.ds(eoff_of(k_idx), CHK2)], valsb)

        def pair(i, carry):
            kA = 2 * i
            kB = 2 * i + 1
            pltpu.sync_copy(col_h.at[pl.ds(eoff_of(kA), CHK2)], colA)
            gA = pltpu.async_copy(sup_h.at[colA], rowsA, semA)
            pltpu.sync_copy(col_h.at[pl.ds(eoff_of(kB), CHK2)], colB)
            gB = pltpu.async_copy(sup_h.at[colB], rowsB, semB)
            stage_rv(kA, rowA)
            gA.wait()
            _scale_rows(rowsA, valsb, NG, CHK2)
            scatA = pltpu.async_copy(rowsA, acc_sp.at[rowA], ssA, add=True)
            stage_rv(kB, rowB)
            gB.wait()
            _scale_rows(rowsB, valsb, NG, CHK2)
            scatA.wait()
            scatB = pltpu.async_copy(rowsB, acc_sp.at[rowB], ssB, add=True)
            scatB.wait()
            return carry
        lax.fori_loop(0, KPT // 2, pair, 0)

        if KPT % 2:
            k = KPT - 1
            pltpu.sync_copy(col_h.at[pl.ds(eoff_of(k), CHK2)], colA)
            gA = pltpu.async_copy(sup_h.at[colA], rowsA, semA)
            stage_rv(k, rowA)
            gA.wait()
            _scale_rows(rowsA, valsb, NG, CHK2)
            pltpu.sync_copy(rowsA, acc_sp.at[rowA], add=True)

        plsc.subcore_barrier()
        _copy_out_slice(acc_sp, part_h, c, s, NPT, TAIL, N)

    mesh = plsc.VectorSubcoreMesh(
        core_axis_name="c", subcore_axis_name="s",
        num_cores=_NSC, num_subcores=_NTEC)
    fn = pl.kernel(
        body,
        out_type=jax.ShapeDtypeStruct((_NSC, N, C), jnp.float32),
        mesh=mesh,
        scratch_types=[
            pltpu.VMEM((CHK2,), jnp.int32),
            pltpu.VMEM((CHK2,), jnp.int32),
            pltpu.VMEM((CHK2,), jnp.int32),
            pltpu.VMEM((CHK2,), jnp.int32),
            pltpu.VMEM((CHK2,), jnp.float32),
            pltpu.VMEM((CHK2, C), jnp.float32),
            pltpu.VMEM((CHK2, C), jnp.float32),
            pltpu.SemaphoreType.DMA,
            pltpu.SemaphoreType.DMA,
            pltpu.SemaphoreType.DMA,
            pltpu.SemaphoreType.DMA,
            pltpu.VMEM_SHARED((N, C), jnp.float32),
        ],
        compiler_params=pltpu.CompilerParams(
            needs_layout_passes=False, use_tc_tiling_on_sc=False),
    )
    return fn(vals, row, col, sup)


def kernel(x, indices, values, size, W1, b1, W2, b2, edge_weights):
    N, _ = x.shape
    E = edge_weights.shape[0]
    H = W1.shape[1]
    C = W2.shape[1]
    row = indices[0]
    col = indices[1]
    ew2 = edge_weights.reshape(E // 128, 128)
    row2 = row.reshape(E // 128, 128)
    size_arr = jnp.full((1, 1), size, jnp.int32)

    sup1, s2 = _tc_dense1(x, W1, b1.reshape(1, H), ew2, row2, size_arr)
    vals, part1 = _sc_layer1(s2.reshape(E), row, col, sup1)
    sup2 = _tc_dense2(part1[0], part1[1], W2, b2.reshape(1, C))
    part2 = _sc_layer2(vals, row, col, sup2)
    return _tc_logsoftmax(part2[0], part2[1])
